# Initial kernel scaffold; baseline (speedup 1.0000x reference)
import jax
import jax.numpy as jnp
from jax.experimental import pallas as pl


def kernel(x, emb_table):
    raise NotImplementedError("write your pallas kernel here")


# speedup vs baseline: 1.9671x; 1.9671x over previous
import jax
import jax.numpy as jnp
from jax import lax
from jax.experimental import pallas as pl
from jax.experimental.pallas import tpu as pltpu
from jax.experimental.pallas import tpu_sc as plsc

BATCH = 16384
EMBED_DIM = 2
NUM_WORKERS = 32
BPW = BATCH // NUM_WORKERS
OPW = BPW * EMBED_DIM
L = 16


def _sc_body(idx_hbm, tab_hbm, out_hbm, idx_v, tab_v, out_v):
    c = lax.axis_index("c")
    s = lax.axis_index("s")
    wid = s * 2 + c
    base = wid * BPW
    pltpu.sync_copy(idx_hbm.at[pl.ds(base, BPW)], idx_v)
    pltpu.sync_copy(tab_hbm, tab_v)
    iota = lax.iota(jnp.int32, L)
    half = iota >> 1
    par = iota & 1
    for k in range(OPW // L):
        jidx = half + (k * (L // 2))
        rows = plsc.load_gather(idx_v, [jidx])
        addr = rows * EMBED_DIM + par
        vals = plsc.load_gather(tab_v, [addr])
        out_v[pl.ds(k * L, L)] = vals
    pltpu.sync_copy(out_v, out_hbm.at[pl.ds(base * EMBED_DIM, OPW)])


def kernel(x, emb_table):
    tab_flat = jnp.concatenate(
        [emb_table.reshape(-1), jnp.zeros((2,), jnp.float32)]
    )
    xi = x.astype(jnp.int32)
    mesh = plsc.VectorSubcoreMesh(core_axis_name="c", subcore_axis_name="s")
    out_flat = pl.kernel(
        _sc_body,
        out_type=jax.ShapeDtypeStruct((BATCH * EMBED_DIM,), jnp.float32),
        mesh=mesh,
        compiler_params=pltpu.CompilerParams(needs_layout_passes=False),
        scratch_types=[
            pltpu.VMEM((BPW,), jnp.int32),
            pltpu.VMEM((8,), jnp.float32),
            pltpu.VMEM((OPW,), jnp.float32),
        ],
    )(xi, tab_flat)
    return out_flat.reshape(BATCH, EMBED_DIM)
